# baseline (speedup 1.0000x reference)
import jax
import jax.numpy as jnp
from jax.experimental import pallas as pl
from jax.experimental.pallas import tpu as pltpu

_N = 10000
_ROWS = 200
_NB = _N // _ROWS
_DEPTH = 4


def _proj_body(feat_ref, w1_ref, u_ref):
    u_ref[...] = jnp.dot(feat_ref[...], w1_ref[...],
                         preferred_element_type=jnp.float32)


def _body(u_ref, adj_ref, b1_ref, w2_ref, b2_ref,
          x1_ref, out_ref, bufs_ref, sems_ref, v_ref):
    i = pl.program_id(0)

    def _copy(k):
        row = (k % _NB) * _ROWS
        return pltpu.make_async_copy(
            adj_ref.at[pl.ds(row, _ROWS), :],
            bufs_ref.at[k % _DEPTH],
            sems_ref.at[k % _DEPTH],
        )

    @pl.when(i == 0)
    def _():
        _copy(0).start()
        _copy(1).start()
        _copy(2).start()

    @pl.when(i + _DEPTH - 1 < 2 * _NB)
    def _():
        _copy(i + _DEPTH - 1).start()

    _copy(i).wait()
    a = bufs_ref[i % _DEPTH]

    @pl.when(i < _NB)
    def _():
        h = jnp.dot(a, u_ref[...], preferred_element_type=jnp.float32)
        x1v = jnp.maximum(h + b1_ref[...], 0.0)
        x1_ref[...] = x1v
        v_ref[pl.ds(i * _ROWS, _ROWS), :] = jnp.dot(
            x1v, w2_ref[...], preferred_element_type=jnp.float32)

    @pl.when(i >= _NB)
    def _():
        h = jnp.dot(a, v_ref[...], preferred_element_type=jnp.float32)
        h = h + b2_ref[...]
        m = jnp.max(h, axis=1, keepdims=True)
        e = jnp.exp(h - m)
        s = jnp.sum(e, axis=1, keepdims=True)
        out_ref[...] = h - m - jnp.log(s)


def kernel(feature, adj, W1, b1, W2, b2):
    f_in = feature.shape[1]
    hid = W1.shape[1]
    dim = W2.shape[1]
    b1r = b1.reshape(1, hid)
    b2r = b2.reshape(1, dim)

    u = pl.pallas_call(
        _proj_body,
        in_specs=[
            pl.BlockSpec((_N, f_in), lambda: (0, 0)),
            pl.BlockSpec((f_in, hid), lambda: (0, 0)),
        ],
        out_specs=pl.BlockSpec((_N, hid), lambda: (0, 0)),
        out_shape=jax.ShapeDtypeStruct((_N, hid), jnp.float32),
    )(feature, W1)

    x1, out = pl.pallas_call(
        _body,
        grid=(2 * _NB,),
        in_specs=[
            pl.BlockSpec((_N, hid), lambda i: (0, 0)),
            pl.BlockSpec(memory_space=pltpu.MemorySpace.HBM),
            pl.BlockSpec((1, hid), lambda i: (0, 0)),
            pl.BlockSpec((hid, dim), lambda i: (0, 0)),
            pl.BlockSpec((1, dim), lambda i: (0, 0)),
        ],
        out_specs=[
            pl.BlockSpec((_ROWS, hid),
                         lambda i: (jnp.where(i < _NB, i, _NB - 1), 0)),
            pl.BlockSpec((_ROWS, dim),
                         lambda i: (jnp.where(i < _NB, 0, i - _NB), 0)),
        ],
        out_shape=[
            jax.ShapeDtypeStruct((_N, hid), jnp.float32),
            jax.ShapeDtypeStruct((_N, dim), jnp.float32),
        ],
        scratch_shapes=[
            pltpu.VMEM((_DEPTH, _ROWS, _N), jnp.float32),
            pltpu.SemaphoreType.DMA((_DEPTH,)),
            pltpu.VMEM((_N, dim), jnp.float32),
        ],
    )(u, adj, b1r, W2, b2r)
    return (x1, out)
